# Initial kernel scaffold; baseline (speedup 1.0000x reference)
import functools
import jax
import jax.numpy as jnp
from jax import lax
from jax.experimental import pallas as pl
from jax.experimental.pallas import tpu as pltpu
from jax.experimental.pallas import tpu_sc as plsc

N = 10000
E = 320000
D = 128
R = 3
NTYPES = 4

NC = 2
NS = 16
NW = NC * NS
EPW = E // NW
CH = 80
NCH = EPW // CH
RPT = N // NS

_MESH = plsc.VectorSubcoreMesh(core_axis_name="c", subcore_axis_name="s",
                               num_cores=NC, num_subcores=NS)


def _worker():
    cid = lax.axis_index("c")
    sid = lax.axis_index("s")
    return cid, sid, sid * NC + cid


@functools.partial(
    pl.kernel,
    out_type=(jax.ShapeDtypeStruct((NC, N, 16), jnp.float32),
              jax.ShapeDtypeStruct((E,), jnp.int32)),
    mesh=_MESH,
    scratch_types=(
        pltpu.VMEM((CH,), jnp.int32),
        pltpu.VMEM((CH,), jnp.int32),
        pltpu.VMEM((CH,), jnp.int32),
        pltpu.VMEM((CH,), jnp.int32),
        pltpu.VMEM((CH, 16), jnp.float32),
        pltpu.VMEM((RPT, 16), jnp.float32),
        pltpu.VMEM_SHARED((N, 16), jnp.float32),
        pltpu.SemaphoreType.DMA,
    ),
)
def _sc_deg_idx(src_hbm, et_hbm, dst_hbm, deg_hbm, g_hbm,
                src_v, et_v, dst_v, g_v, ones_v, zero_v, deg_sh, sem):
    cid, sid, wid = _worker()

    def fill(i, _):
        ones_v[i] = jnp.full((16,), 1.0, jnp.float32)
        return 0
    lax.fori_loop(0, CH, fill, 0)

    def zrow(i, _):
        zero_v[i] = jnp.zeros((16,), jnp.float32)
        return 0
    lax.fori_loop(0, RPT, zrow, 0)
    pltpu.sync_copy(zero_v, deg_sh.at[pl.ds(sid * RPT, RPT)])
    plsc.subcore_barrier()

    def chunk(c, _):
        base = wid * EPW + c * CH
        pltpu.sync_copy(src_hbm.at[pl.ds(base, CH)], src_v)
        pltpu.sync_copy(et_hbm.at[pl.ds(base, CH)], et_v)
        pltpu.sync_copy(dst_hbm.at[pl.ds(base, CH)], dst_v)
        for j in range(CH // 16):
            sl = pl.ds(j * 16, 16)
            g_v[sl] = et_v[sl] * N + src_v[sl]
        pltpu.sync_copy(g_v, g_hbm.at[pl.ds(base, CH)])
        pltpu.sync_copy(ones_v, deg_sh.at[dst_v], add=True)
        return 0
    lax.fori_loop(0, NCH, chunk, 0)
    plsc.subcore_barrier()

    rows = pl.ds(sid * RPT, RPT)
    pltpu.sync_copy(deg_sh.at[rows], deg_hbm.at[cid].at[rows])


@functools.partial(
    pl.kernel,
    out_type=jax.ShapeDtypeStruct((NC, N, D), jnp.float32),
    mesh=_MESH,
    scratch_types=(
        pltpu.VMEM((CH,), jnp.int32),
        pltpu.VMEM((CH,), jnp.int32),
        pltpu.VMEM((CH, D), jnp.float32),
        pltpu.VMEM((125, D), jnp.float32),
        pltpu.VMEM_SHARED((N, D), jnp.float32),
        pltpu.SemaphoreType.DMA,
    ),
)
def _sc_segsum(y_hbm, g_hbm, dst_hbm, acc_hbm,
               g_v, dst_v, rows_v, zero_v, acc_sh, sem):
    cid, sid, wid = _worker()

    def zrow(i, _):
        for j in range(D // 16):
            zero_v[i, pl.ds(j * 16, 16)] = jnp.zeros((16,), jnp.float32)
        return 0
    lax.fori_loop(0, 125, zrow, 0)
    for k in range(RPT // 125):
        pltpu.sync_copy(zero_v, acc_sh.at[pl.ds(sid * RPT + k * 125, 125)])
    plsc.subcore_barrier()

    def chunk(c, _):
        base = wid * EPW + c * CH
        pltpu.sync_copy(g_hbm.at[pl.ds(base, CH)], g_v)
        pltpu.sync_copy(dst_hbm.at[pl.ds(base, CH)], dst_v)
        pltpu.async_copy(y_hbm.at[g_v], rows_v, sem).wait()
        pltpu.sync_copy(rows_v, acc_sh.at[dst_v], add=True)
        return 0
    lax.fori_loop(0, NCH, chunk, 0)
    plsc.subcore_barrier()

    rows = pl.ds(sid * RPT, RPT)
    pltpu.sync_copy(acc_sh.at[rows], acc_hbm.at[cid].at[rows])


_BN = 1000
_GRID = N // _BN


def _enc_body(z_ref, nt_ref, emb_ref, w1_ref, b1_ref, w2_ref, b2_ref, x_ref):
    zc = z_ref[...]
    oh = (lax.broadcasted_iota(jnp.int32, (_BN, 100), 1) == zc)
    zf = jnp.dot(oh.astype(jnp.float32), emb_ref[...],
                 preferred_element_type=jnp.float32)
    ntc = nt_ref[...]
    acc = jnp.zeros((_BN, D), jnp.float32)
    for t in range(NTYPES):
        h1 = jnp.maximum(
            jnp.dot(zf, w1_ref[t], preferred_element_type=jnp.float32)
            + b1_ref[t], 0.0)
        h2 = (jnp.dot(h1, w2_ref[t], preferred_element_type=jnp.float32)
              + b2_ref[t])
        acc = acc + jnp.where(ntc == t, 1.0, 0.0) * h2
    x_ref[...] = acc


_encoder = pl.pallas_call(
    _enc_body,
    grid=(_GRID,),
    in_specs=[
        pl.BlockSpec((_BN, 1), lambda i: (i, 0)),
        pl.BlockSpec((_BN, 1), lambda i: (i, 0)),
        pl.BlockSpec((100, D), lambda i: (0, 0)),
        pl.BlockSpec((NTYPES, D, D), lambda i: (0, 0, 0)),
        pl.BlockSpec((NTYPES, 1, D), lambda i: (0, 0, 0)),
        pl.BlockSpec((NTYPES, D, D), lambda i: (0, 0, 0)),
        pl.BlockSpec((NTYPES, 1, D), lambda i: (0, 0, 0)),
    ],
    out_specs=pl.BlockSpec((_BN, D), lambda i: (i, 0)),
    out_shape=jax.ShapeDtypeStruct((N, D), jnp.float32),
)


def _rel_body(x_ref, relw_ref, linw_ref, linb_ref, y_ref, zlin_ref):
    xb = x_ref[...]
    for r in range(R):
        y_ref[r] = jnp.dot(xb, relw_ref[r], preferred_element_type=jnp.float32)
    zlin_ref[...] = (jnp.dot(xb, linw_ref[...],
                             preferred_element_type=jnp.float32)
                     + linb_ref[...])


_relmm = pl.pallas_call(
    _rel_body,
    grid=(_GRID,),
    in_specs=[
        pl.BlockSpec((_BN, D), lambda i: (i, 0)),
        pl.BlockSpec((R, D, D), lambda i: (0, 0, 0)),
        pl.BlockSpec((D, D), lambda i: (0, 0)),
        pl.BlockSpec((1, D), lambda i: (0, 0)),
    ],
    out_specs=[
        pl.BlockSpec((R, _BN, D), lambda i: (0, i, 0)),
        pl.BlockSpec((_BN, D), lambda i: (i, 0)),
    ],
    out_shape=[
        jax.ShapeDtypeStruct((R, N, D), jnp.float32),
        jax.ShapeDtypeStruct((N, D), jnp.float32),
    ],
)


def _ln_body(zlin_ref, acc_ref, deg_ref, g_ref, b_ref, x_ref):
    out = acc_ref[0] + acc_ref[1]
    deg = jnp.maximum(deg_ref[0, :, 0:1] + deg_ref[1, :, 0:1], 1.0)
    t = zlin_ref[...] + out / deg
    mu = jnp.mean(t, axis=1, keepdims=True)
    var = jnp.mean((t - mu) ** 2, axis=1, keepdims=True)
    x_ref[...] = (t - mu) * lax.rsqrt(var + 1e-5) * g_ref[...] + b_ref[...]


_lnorm = pl.pallas_call(
    _ln_body,
    grid=(_GRID,),
    in_specs=[
        pl.BlockSpec((_BN, D), lambda i: (i, 0)),
        pl.BlockSpec((NC, _BN, D), lambda i: (0, i, 0)),
        pl.BlockSpec((NC, _BN, 16), lambda i: (0, i, 0)),
        pl.BlockSpec((1, D), lambda i: (0, 0)),
        pl.BlockSpec((1, D), lambda i: (0, 0)),
    ],
    out_specs=pl.BlockSpec((_BN, D), lambda i: (i, 0)),
    out_shape=jax.ShapeDtypeStruct((N, D), jnp.float32),
)


def _pool_body(x_ref, w_ref, b_ref, o_ref):
    pooled = jnp.mean(x_ref[...], axis=0, keepdims=True)
    o_ref[...] = (jnp.dot(pooled, w_ref[...],
                          preferred_element_type=jnp.float32) + b_ref[...])


_pool = pl.pallas_call(
    _pool_body,
    out_shape=jax.ShapeDtypeStruct((1, 1), jnp.float32),
)


@jax.jit
def kernel(z_embed, enc_W1, enc_b1, enc_W2, enc_b2, lin_W, lin_b, rel_W,
           ln_g, ln_b, reg_W, reg_b, z, node_type, edge_index, edge_type):
    src = edge_index[0].astype(jnp.int32)
    dst = edge_index[1].astype(jnp.int32)
    et = edge_type.astype(jnp.int32)

    x = _encoder(z.astype(jnp.int32).reshape(N, 1),
                 node_type.astype(jnp.int32).reshape(N, 1),
                 z_embed, enc_W1, enc_b1.reshape(NTYPES, 1, D),
                 enc_W2, enc_b2.reshape(NTYPES, 1, D))

    deg2, g = _sc_deg_idx(src, et, dst)

    for l in range(2):
        y, zlin = _relmm(x, rel_W[l], lin_W[l], lin_b[l].reshape(1, D))
        acc2 = _sc_segsum(y.reshape(R * N, D), g, dst)
        x = _lnorm(zlin, acc2, deg2,
                   ln_g[l].reshape(1, D), ln_b[l].reshape(1, D))

    out = _pool(x, reg_W, reg_b.reshape(1, 1))
    return out.reshape(1)


# speedup vs baseline: 6.7523x; 6.7523x over previous
import functools
import jax
import jax.numpy as jnp
from jax import lax
from jax.experimental import pallas as pl
from jax.experimental.pallas import tpu as pltpu
from jax.experimental.pallas import tpu_sc as plsc

N = 10000
E = 320000
D = 128
R = 3
NTYPES = 4

NC = 2
NS = 16
NW = NC * NS
EPW = E // NW
CH = 80
NCH = EPW // CH


def _worker():
    cid = lax.axis_index("c")
    sid = lax.axis_index("s")
    return cid, sid, sid * NC + cid


def _sc_deg_body(src_hbm, et_hbm, dst_hbm, znd_hbm, deg_hbm, g_hbm,
                 src_v, et_v, dst_v, g_v, ones_v, deg_sh):
    cid, sid, wid = _worker()

    def fill(i, _):
        for j in range(D // 16):
            ones_v[i, pl.ds(j * 16, 16)] = jnp.full((16,), 1.0, jnp.float32)
        return 0
    lax.fori_loop(0, CH, fill, 0)

    @pl.when(sid == 0)
    def _():
        pltpu.sync_copy(znd_hbm, deg_sh)
    plsc.subcore_barrier()

    def chunk(c, _):
        base = wid * EPW + c * CH
        pltpu.sync_copy(src_hbm.at[pl.ds(base, CH)], src_v)
        pltpu.sync_copy(et_hbm.at[pl.ds(base, CH)], et_v)
        pltpu.sync_copy(dst_hbm.at[pl.ds(base, CH)], dst_v)
        for j in range(CH // 16):
            sl = pl.ds(j * 16, 16)
            g_v[sl] = et_v[sl] * N + src_v[sl]
        pltpu.sync_copy(g_v, g_hbm.at[pl.ds(base, CH)])
        pltpu.sync_copy(ones_v, deg_sh.at[dst_v], add=True)
        return 0
    lax.fori_loop(0, NCH, chunk, 0)
    plsc.subcore_barrier()

    @pl.when(sid == 0)
    def _():
        pltpu.sync_copy(deg_sh, deg_hbm.at[cid])


def _sc_seg_body(y_hbm, g_hbm, dst_hbm, znd_hbm, acc_hbm,
                 g_v, dst_v, rows_v, acc_sh, sem):
    cid, sid, wid = _worker()

    @pl.when(sid == 0)
    def _():
        pltpu.sync_copy(znd_hbm, acc_sh)
    plsc.subcore_barrier()

    def chunk(c, _):
        base = wid * EPW + c * CH
        pltpu.sync_copy(g_hbm.at[pl.ds(base, CH)], g_v)
        pltpu.sync_copy(dst_hbm.at[pl.ds(base, CH)], dst_v)
        pltpu.async_copy(y_hbm.at[g_v], rows_v, sem).wait()
        pltpu.sync_copy(rows_v, acc_sh.at[dst_v], add=True)
        return 0
    lax.fori_loop(0, NCH, chunk, 0)
    plsc.subcore_barrier()

    @pl.when(sid == 0)
    def _():
        pltpu.sync_copy(acc_sh, acc_hbm.at[cid])


@functools.cache
def _sc_kernels():
    mesh = plsc.VectorSubcoreMesh(core_axis_name="c", subcore_axis_name="s",
                                  num_cores=NC, num_subcores=NS)
    deg_fn = pl.kernel(
        _sc_deg_body,
        out_type=(jax.ShapeDtypeStruct((NC, N, D), jnp.float32),
                  jax.ShapeDtypeStruct((E,), jnp.int32)),
        mesh=mesh,
        scratch_types=(
            pltpu.VMEM((CH,), jnp.int32),
            pltpu.VMEM((CH,), jnp.int32),
            pltpu.VMEM((CH,), jnp.int32),
            pltpu.VMEM((CH,), jnp.int32),
            pltpu.VMEM((CH, D), jnp.float32),
            pltpu.VMEM_SHARED((N, D), jnp.float32),
        ),
    )
    seg_fn = pl.kernel(
        _sc_seg_body,
        out_type=jax.ShapeDtypeStruct((NC, N, D), jnp.float32),
        mesh=mesh,
        scratch_types=(
            pltpu.VMEM((CH,), jnp.int32),
            pltpu.VMEM((CH,), jnp.int32),
            pltpu.VMEM((CH, D), jnp.float32),
            pltpu.VMEM_SHARED((N, D), jnp.float32),
            pltpu.SemaphoreType.DMA,
        ),
    )
    return deg_fn, seg_fn


_BN = 1000
_GRID = N // _BN


def _enc_body(z_ref, nt_ref, emb_ref, w1_ref, b1_ref, w2_ref, b2_ref, x_ref):
    zc = z_ref[...]
    oh = (lax.broadcasted_iota(jnp.int32, (_BN, 100), 1) == zc)
    zf = jnp.dot(oh.astype(jnp.float32), emb_ref[...],
                 preferred_element_type=jnp.float32)
    ntc = nt_ref[...]
    acc = jnp.zeros((_BN, D), jnp.float32)
    for t in range(NTYPES):
        h1 = jnp.maximum(
            jnp.dot(zf, w1_ref[t], preferred_element_type=jnp.float32)
            + b1_ref[t], 0.0)
        h2 = (jnp.dot(h1, w2_ref[t], preferred_element_type=jnp.float32)
              + b2_ref[t])
        acc = acc + jnp.where(ntc == t, 1.0, 0.0) * h2
    x_ref[...] = acc


def _rel_body(x_ref, relw_ref, linw_ref, linb_ref, y_ref, zlin_ref):
    xb = x_ref[...]
    for r in range(R):
        y_ref[r] = jnp.dot(xb, relw_ref[r], preferred_element_type=jnp.float32)
    zlin_ref[...] = (jnp.dot(xb, linw_ref[...],
                             preferred_element_type=jnp.float32)
                     + linb_ref[...])


def _ln_body(zlin_ref, acc_ref, deg_ref, g_ref, b_ref, x_ref):
    out = acc_ref[0] + acc_ref[1]
    deg = jnp.maximum(deg_ref[0, :, 0:1] + deg_ref[1, :, 0:1], 1.0)
    t = zlin_ref[...] + out / deg
    mu = jnp.mean(t, axis=1, keepdims=True)
    var = jnp.mean((t - mu) ** 2, axis=1, keepdims=True)
    x_ref[...] = (t - mu) * lax.rsqrt(var + 1e-5) * g_ref[...] + b_ref[...]


def _pool_body(x_ref, w_ref, b_ref, o_ref):
    pooled = jnp.mean(x_ref[...], axis=0, keepdims=True)
    o_ref[...] = (jnp.dot(pooled, w_ref[...],
                          preferred_element_type=jnp.float32) + b_ref[...])


@functools.cache
def _tc_kernels(interpret=False):
    encoder = pl.pallas_call(
        _enc_body,
        grid=(_GRID,),
        in_specs=[
            pl.BlockSpec((_BN, 1), lambda i: (i, 0)),
            pl.BlockSpec((_BN, 1), lambda i: (i, 0)),
            pl.BlockSpec((100, D), lambda i: (0, 0)),
            pl.BlockSpec((NTYPES, D, D), lambda i: (0, 0, 0)),
            pl.BlockSpec((NTYPES, 1, D), lambda i: (0, 0, 0)),
            pl.BlockSpec((NTYPES, D, D), lambda i: (0, 0, 0)),
            pl.BlockSpec((NTYPES, 1, D), lambda i: (0, 0, 0)),
        ],
        out_specs=pl.BlockSpec((_BN, D), lambda i: (i, 0)),
        out_shape=jax.ShapeDtypeStruct((N, D), jnp.float32),
        interpret=interpret,
    )
    relmm = pl.pallas_call(
        _rel_body,
        grid=(_GRID,),
        in_specs=[
            pl.BlockSpec((_BN, D), lambda i: (i, 0)),
            pl.BlockSpec((R, D, D), lambda i: (0, 0, 0)),
            pl.BlockSpec((D, D), lambda i: (0, 0)),
            pl.BlockSpec((1, D), lambda i: (0, 0)),
        ],
        out_specs=[
            pl.BlockSpec((R, _BN, D), lambda i: (0, i, 0)),
            pl.BlockSpec((_BN, D), lambda i: (i, 0)),
        ],
        out_shape=[
            jax.ShapeDtypeStruct((R, N, D), jnp.float32),
            jax.ShapeDtypeStruct((N, D), jnp.float32),
        ],
        interpret=interpret,
    )
    lnorm = pl.pallas_call(
        _ln_body,
        grid=(_GRID,),
        in_specs=[
            pl.BlockSpec((_BN, D), lambda i: (i, 0)),
            pl.BlockSpec((NC, _BN, D), lambda i: (0, i, 0)),
            pl.BlockSpec((NC, _BN, D), lambda i: (0, i, 0)),
            pl.BlockSpec((1, D), lambda i: (0, 0)),
            pl.BlockSpec((1, D), lambda i: (0, 0)),
        ],
        out_specs=pl.BlockSpec((_BN, D), lambda i: (i, 0)),
        out_shape=jax.ShapeDtypeStruct((N, D), jnp.float32),
        interpret=interpret,
    )
    pool = pl.pallas_call(
        _pool_body,
        out_shape=jax.ShapeDtypeStruct((1, 1), jnp.float32),
        interpret=interpret,
    )
    return encoder, relmm, lnorm, pool


@jax.jit
def kernel(z_embed, enc_W1, enc_b1, enc_W2, enc_b2, lin_W, lin_b, rel_W,
           ln_g, ln_b, reg_W, reg_b, z, node_type, edge_index, edge_type):
    encoder, relmm, lnorm, pool = _tc_kernels()
    deg_fn, seg_fn = _sc_kernels()

    src = edge_index[0].astype(jnp.int32)
    dst = edge_index[1].astype(jnp.int32)
    et = edge_type.astype(jnp.int32)
    znd = jnp.zeros((N, D), jnp.float32)

    x = encoder(z.astype(jnp.int32).reshape(N, 1),
                node_type.astype(jnp.int32).reshape(N, 1),
                z_embed, enc_W1, enc_b1.reshape(NTYPES, 1, D),
                enc_W2, enc_b2.reshape(NTYPES, 1, D))

    deg2, g = deg_fn(src, et, dst, znd)

    for l in range(2):
        y, zlin = relmm(x, rel_W[l], lin_W[l], lin_b[l].reshape(1, D))
        acc2 = seg_fn(y.reshape(R * N, D), g, dst, znd)
        x = lnorm(zlin, acc2, deg2,
                  ln_g[l].reshape(1, D), ln_b[l].reshape(1, D))

    out = pool(x, reg_W, reg_b.reshape(1, 1))
    return out.reshape(1)
